# baseline (speedup 1.0000x reference)
import functools

import jax
import jax.numpy as jnp
from jax import lax
from jax.experimental import pallas as pl
from jax.experimental.pallas import tpu as pltpu
from jax.experimental.pallas import tpu_sc as plsc

N = 100000
DIM_IN = 128
DIM_H = 96
DIM_PE = 32
NUM_TYPES = 1000

NW = 32
CH = 128
CPW = 25
N_PAD = NW * CPW * CH
LAST_FULL = N // CH - 1
GRP = 25
CH2 = GRP * CH
GPW = CPW // GRP
LAST_FULL2 = N // CH2 - 1
TAIL2 = N - (LAST_FULL2 + 1) * CH2


def _sc_scatter_body(idx_hbm, table_hbm, out_ref, idx_v, rows_v, sem):
    wid = lax.axis_index("s") * 2 + lax.axis_index("c")
    pltpu.sync_copy(idx_hbm.at[wid], idx_v)

    copies = [
        pltpu.async_copy(table_hbm.at[idx_v.at[j]],
                         rows_v.at[pl.ds(j * CH, CH)], sem)
        for j in range(CPW)
    ]
    for cp in copies:
        cp.wait()

    r0 = wid * CH2

    @pl.when(wid * GPW <= LAST_FULL2)
    def _full():
        pltpu.sync_copy(rows_v,
                        out_ref.at[pl.ds(r0, CH2), pl.ds(DIM_H, DIM_PE)])

    @pl.when(wid * GPW == LAST_FULL2 + 1)
    def _tail():
        pltpu.sync_copy(rows_v.at[pl.ds(0, TAIL2)],
                        out_ref.at[pl.ds(r0, TAIL2), pl.ds(DIM_H, DIM_PE)])


@functools.cache
def _sc_scatter():
    return pl.kernel(
        _sc_scatter_body,
        out_type=(),
        mesh=plsc.VectorSubcoreMesh(core_axis_name="c", subcore_axis_name="s"),
        scratch_types=[
            pltpu.VMEM((CPW, CH), jnp.int32),
            pltpu.VMEM((CH2, DIM_PE), jnp.float32),
            pltpu.SemaphoreType.DMA,
        ],
        compiler_params=pltpu.CompilerParams(use_tc_tiling_on_sc=False),
    )


def _tc_body(x_ref, w_ref, b_ref, out_ref):
    h = jnp.dot(x_ref[:], w_ref[:], preferred_element_type=jnp.float32)
    out_ref[:, 0:DIM_H] = h + b_ref[:]


BLK = 10000


def _tc_matmul(x, W, b2):
    return pl.pallas_call(
        _tc_body,
        grid=(N // BLK,),
        in_specs=[
            pl.BlockSpec((BLK, DIM_IN), lambda i: (i, 0)),
            pl.BlockSpec((DIM_IN, DIM_H), lambda i: (0, 0)),
            pl.BlockSpec((1, DIM_H), lambda i: (0, 0)),
        ],
        out_specs=pl.BlockSpec((BLK, DIM_IN), lambda i: (i, 0)),
        out_shape=jax.ShapeDtypeStruct((N, DIM_IN), jnp.float32),
        compiler_params=pltpu.CompilerParams(
            dimension_semantics=("parallel",),
        ),
    )(x, W, b2)


def kernel(x, WLTag, W, b, emb_table):
    idx = WLTag.reshape(-1).astype(jnp.int32)
    idx = jnp.pad(idx, (0, N_PAD - N)).reshape(NW, CPW, CH)
    out_h = _tc_matmul(x, W, b.reshape(1, DIM_H))
    out_ref = jax.new_ref(out_h)
    _sc_scatter()(idx, emb_table, out_ref)
    return jax.freeze(out_ref)


# speedup vs baseline: 1.2951x; 1.2951x over previous
import functools

import jax
import jax.numpy as jnp
from jax import lax
from jax.experimental import pallas as pl
from jax.experimental.pallas import tpu as pltpu
from jax.experimental.pallas import tpu_sc as plsc

N = 100000
DIM_IN = 128
DIM_H = 96
DIM_PE = 32
NUM_TYPES = 1000

NW = 32
CH = 128
CPW = 25
N_PAD = NW * CPW * CH
LAST_FULL = N // CH - 1
GRP = 25
CH2 = GRP * CH
GPW = CPW // GRP
LAST_FULL2 = N // CH2 - 1
TAIL2 = N - (LAST_FULL2 + 1) * CH2


def _sc_scatter_body(idx_hbm, table_hbm, out_ref, idx_v, rows_v, sem):
    wid = lax.axis_index("s") * 2 + lax.axis_index("c")
    pltpu.sync_copy(idx_hbm.at[wid], idx_v)

    for j in range(CPW):
        c = wid * CPW + j

        @pl.when(c <= LAST_FULL + 1)
        def _gather():
            pltpu.async_copy(table_hbm.at[idx_v.at[j]],
                             rows_v.at[pl.ds(j * CH, CH)], sem).wait()

    r0 = wid * CH2

    @pl.when(wid <= LAST_FULL2)
    def _full():
        pltpu.sync_copy(rows_v,
                        out_ref.at[pl.ds(r0, CH2), pl.ds(DIM_H, DIM_PE)])

    @pl.when(wid == LAST_FULL2 + 1)
    def _tail():
        pltpu.sync_copy(rows_v.at[pl.ds(0, TAIL2)],
                        out_ref.at[pl.ds(r0, TAIL2), pl.ds(DIM_H, DIM_PE)])


@functools.cache
def _sc_scatter():
    return pl.kernel(
        _sc_scatter_body,
        out_type=(),
        mesh=plsc.VectorSubcoreMesh(core_axis_name="c", subcore_axis_name="s"),
        scratch_types=[
            pltpu.VMEM((CPW, CH), jnp.int32),
            pltpu.VMEM((CH2, DIM_PE), jnp.float32),
            pltpu.SemaphoreType.DMA,
        ],
        compiler_params=pltpu.CompilerParams(use_tc_tiling_on_sc=False),
    )


def _tc_body(x_ref, w_ref, b_ref, out_ref):
    h = jnp.dot(x_ref[:], w_ref[:], preferred_element_type=jnp.float32)
    out_ref[:, 0:DIM_H] = h + b_ref[:]


BLK = 10000


def _tc_matmul(x, W, b2):
    return pl.pallas_call(
        _tc_body,
        grid=(N // BLK,),
        in_specs=[
            pl.BlockSpec((BLK, DIM_IN), lambda i: (i, 0)),
            pl.BlockSpec((DIM_IN, DIM_H), lambda i: (0, 0)),
            pl.BlockSpec((1, DIM_H), lambda i: (0, 0)),
        ],
        out_specs=pl.BlockSpec((BLK, DIM_IN), lambda i: (i, 0)),
        out_shape=jax.ShapeDtypeStruct((N, DIM_IN), jnp.float32),
        compiler_params=pltpu.CompilerParams(
            dimension_semantics=("parallel",),
        ),
    )(x, W, b2)


def kernel(x, WLTag, W, b, emb_table):
    idx = WLTag.reshape(-1).astype(jnp.int32)
    idx = jnp.pad(idx, (0, N_PAD - N)).reshape(NW, CPW, CH)
    out_h = _tc_matmul(x, W, b.reshape(1, DIM_H))
    out_ref = jax.new_ref(out_h)
    _sc_scatter()(idx, emb_table, out_ref)
    return jax.freeze(out_ref)


# speedup vs baseline: 1.3154x; 1.0156x over previous
import functools

import jax
import jax.numpy as jnp
from jax import lax
from jax.experimental import pallas as pl
from jax.experimental.pallas import tpu as pltpu
from jax.experimental.pallas import tpu_sc as plsc

N = 100000
DIM_IN = 128
DIM_H = 96
DIM_PE = 32
NUM_TYPES = 1000

NW = 32
CH = 128
CPW = 25
N_PAD = NW * CPW * CH
LAST_FULL = N // CH - 1
GRP = 25
CH2 = GRP * CH
GPW = CPW // GRP
LAST_FULL2 = N // CH2 - 1
TAIL2 = N - (LAST_FULL2 + 1) * CH2


def _sc_scatter_body(idx_hbm, table_hbm, out_ref, idx_v, rows_v, sem):
    wid = lax.axis_index("s") * 2 + lax.axis_index("c")
    pltpu.sync_copy(idx_hbm.at[wid], idx_v)

    for j in range(CPW):
        c = wid * CPW + j

        @pl.when(c <= LAST_FULL + 1)
        def _gather():
            pltpu.async_copy(table_hbm.at[idx_v.at[j]],
                             rows_v.at[pl.ds(j * CH, CH)], sem).wait()

    r0 = wid * CH2

    @pl.when(wid <= LAST_FULL2)
    def _full():
        pltpu.sync_copy(rows_v,
                        out_ref.at[pl.ds(r0, CH2), pl.ds(DIM_H, DIM_PE)])

    @pl.when(wid == LAST_FULL2 + 1)
    def _tail():
        pltpu.sync_copy(rows_v.at[pl.ds(0, TAIL2)],
                        out_ref.at[pl.ds(r0, TAIL2), pl.ds(DIM_H, DIM_PE)])


@functools.cache
def _sc_scatter():
    return pl.kernel(
        _sc_scatter_body,
        out_type=(),
        mesh=plsc.VectorSubcoreMesh(core_axis_name="c", subcore_axis_name="s"),
        scratch_types=[
            pltpu.VMEM((CPW, CH), jnp.int32),
            pltpu.VMEM((CH2, DIM_PE), jnp.float32),
            pltpu.SemaphoreType.DMA,
        ],
        compiler_params=pltpu.CompilerParams(use_tc_tiling_on_sc=False),
    )


def _tc_body(x_ref, w_ref, b_ref, out_ref):
    h = jnp.dot(x_ref[:], w_ref[:], preferred_element_type=jnp.float32)
    out_ref[:, 0:DIM_H] = h + b_ref[:]


BLK = 20000


def _tc_matmul(x, W, b2):
    return pl.pallas_call(
        _tc_body,
        grid=(N // BLK,),
        in_specs=[
            pl.BlockSpec((BLK, DIM_IN), lambda i: (i, 0)),
            pl.BlockSpec((DIM_IN, DIM_H), lambda i: (0, 0)),
            pl.BlockSpec((1, DIM_H), lambda i: (0, 0)),
        ],
        out_specs=pl.BlockSpec((BLK, DIM_IN), lambda i: (i, 0)),
        out_shape=jax.ShapeDtypeStruct((N, DIM_IN), jnp.float32),
        compiler_params=pltpu.CompilerParams(
            dimension_semantics=("parallel",),
        ),
    )(x, W, b2)


def kernel(x, WLTag, W, b, emb_table):
    idx = WLTag.reshape(-1).astype(jnp.int32)
    idx = jnp.pad(idx, (0, N_PAD - N)).reshape(NW, CPW, CH)
    out_h = _tc_matmul(x, W, b.reshape(1, DIM_H))
    out_ref = jax.new_ref(out_h)
    _sc_scatter()(idx, emb_table, out_ref)
    return jax.freeze(out_ref)


# speedup vs baseline: 1.6383x; 1.2455x over previous
import functools

import jax
import jax.numpy as jnp
from jax import lax
from jax.experimental import pallas as pl
from jax.experimental.pallas import tpu as pltpu
from jax.experimental.pallas import tpu_sc as plsc

N = 100000
DIM_IN = 128
DIM_H = 96
DIM_PE = 32
NUM_TYPES = 1000

NW = 32
CH = 128
CPW = 25
N_PAD = NW * CPW * CH
LAST_FULL = N // CH - 1
GRP = 25
CH2 = GRP * CH
GPW = CPW // GRP
LAST_FULL2 = N // CH2 - 1
TAIL2 = N - (LAST_FULL2 + 1) * CH2


def _sc_scatter_body(idx_hbm, table_hbm, out_ref, idx_v, rows_v, table_sp, sem):
    sid = lax.axis_index("s")
    wid = sid * 2 + lax.axis_index("c")

    @pl.when(sid == 0)
    def _stage():
        pltpu.sync_copy(table_hbm, table_sp)

    pltpu.sync_copy(idx_hbm.at[wid], idx_v)
    plsc.subcore_barrier()

    for j in range(CPW):
        c = wid * CPW + j

        @pl.when(c <= LAST_FULL + 1)
        def _gather():
            pltpu.async_copy(table_sp.at[idx_v.at[j]],
                             rows_v.at[pl.ds(j * CH, CH)], sem).wait()

    r0 = wid * CH2

    @pl.when(wid <= LAST_FULL2)
    def _full():
        pltpu.sync_copy(rows_v,
                        out_ref.at[pl.ds(r0, CH2), pl.ds(DIM_H, DIM_PE)])

    @pl.when(wid == LAST_FULL2 + 1)
    def _tail():
        pltpu.sync_copy(rows_v.at[pl.ds(0, TAIL2)],
                        out_ref.at[pl.ds(r0, TAIL2), pl.ds(DIM_H, DIM_PE)])


@functools.cache
def _sc_scatter():
    return pl.kernel(
        _sc_scatter_body,
        out_type=(),
        mesh=plsc.VectorSubcoreMesh(core_axis_name="c", subcore_axis_name="s"),
        scratch_types=[
            pltpu.VMEM((CPW, CH), jnp.int32),
            pltpu.VMEM((CH2, DIM_PE), jnp.float32),
            pltpu.VMEM_SHARED((NUM_TYPES, DIM_PE), jnp.float32),
            pltpu.SemaphoreType.DMA,
        ],
        compiler_params=pltpu.CompilerParams(use_tc_tiling_on_sc=False),
    )


def _tc_body(x_ref, w_ref, b_ref, out_ref):
    h = jnp.dot(x_ref[:], w_ref[:], preferred_element_type=jnp.float32)
    out_ref[:, 0:DIM_H] = h + b_ref[:]


BLK = 20000


def _tc_matmul(x, W, b2):
    return pl.pallas_call(
        _tc_body,
        grid=(N // BLK,),
        in_specs=[
            pl.BlockSpec((BLK, DIM_IN), lambda i: (i, 0)),
            pl.BlockSpec((DIM_IN, DIM_H), lambda i: (0, 0)),
            pl.BlockSpec((1, DIM_H), lambda i: (0, 0)),
        ],
        out_specs=pl.BlockSpec((BLK, DIM_IN), lambda i: (i, 0)),
        out_shape=jax.ShapeDtypeStruct((N, DIM_IN), jnp.float32),
        compiler_params=pltpu.CompilerParams(
            dimension_semantics=("parallel",),
        ),
    )(x, W, b2)


def kernel(x, WLTag, W, b, emb_table):
    idx = WLTag.reshape(-1).astype(jnp.int32)
    idx = jnp.pad(idx, (0, N_PAD - N)).reshape(NW, CPW, CH)
    out_h = _tc_matmul(x, W, b.reshape(1, DIM_H))
    out_ref = jax.new_ref(out_h)
    _sc_scatter()(idx, emb_table, out_ref)
    return jax.freeze(out_ref)
